# baseline (speedup 1.0000x reference)
import functools
import math

import jax
import jax.numpy as jnp
from jax import lax
from jax.experimental import pallas as pl
from jax.experimental.pallas import tpu as pltpu
from jax.experimental.pallas import tpu_sc as plsc

_CUTOFF = 5.0


def _zcol_sc(z, col, N, E):
    NW = 32
    EPW = E // NW
    mesh = plsc.VectorSubcoreMesh(core_axis_name="c", subcore_axis_name="s")

    @functools.partial(
        pl.kernel,
        out_type=jax.ShapeDtypeStruct((E,), jnp.int32),
        mesh=mesh,
        scratch_types=[pltpu.VMEM((N,), jnp.int32),
                       pltpu.VMEM((EPW,), jnp.int32),
                       pltpu.VMEM((EPW,), jnp.int32)],
        compiler_params=pltpu.CompilerParams(needs_layout_passes=False),
    )
    def k(z_hbm, col_hbm, zcol_hbm, z_v, col_v, out_v):
        wid = lax.axis_index("c") * 16 + lax.axis_index("s")
        base = wid * EPW
        pltpu.sync_copy(z_hbm, z_v)
        pltpu.sync_copy(col_hbm.at[pl.ds(base, EPW)], col_v)

        def body(j, carry):
            idx = col_v[pl.ds(j * 16, 16)]
            out_v[pl.ds(j * 16, 16)] = plsc.load_gather(z_v, [idx])
            return carry

        lax.fori_loop(0, EPW // 16, body, 0)
        pltpu.sync_copy(out_v, zcol_hbm.at[pl.ds(base, EPW)])

    return k(z, col)


def _encode_tc(ew128, zc128, EB):
    def body(ew_ref, zc_ref, v_ref):
        ew = ew_ref[...]
        c = 0.5 * (jnp.cos(ew * (math.pi / _CUTOFF)) + 1.0)
        c = jnp.where(ew < _CUTOFF, c, 0.0)
        v_ref[...] = zc_ref[...].astype(jnp.float32) + (0.25 + 0.5 * c)

    return pl.pallas_call(
        body,
        out_shape=jax.ShapeDtypeStruct((EB, 128), jnp.float32),
    )(ew128, zc128)


def _msg_tc(v8, ea2, W2, b2p, emb2, EH, R, H, ZP, OFFB):
    BE = 3200
    NB = EH // BE

    def body(v_ref, ea_ref, w2_ref, b_ref, emb_ref, out_ref):
        v = v_ref[...]
        v3 = lax.broadcast_in_dim(v, (BE // 8, 8, ZP), (0, 1))
        zcf3 = jnp.floor(v3)
        c2 = jnp.reshape((v3 - zcf3 - 0.25) * 2.0, (BE // 2, 2 * ZP))
        onehot = jnp.reshape(
            (zcf3.astype(jnp.int32) ==
             lax.broadcasted_iota(jnp.int32, (BE // 8, 8, ZP), 2)
             ).astype(jnp.float32), (BE // 2, 2 * ZP))
        mm = jnp.dot(ea_ref[...], w2_ref[...], preferred_element_type=jnp.float32)
        g = jnp.dot(onehot, emb_ref[...], preferred_element_type=jnp.float32)
        out_ref[...] = (mm + b_ref[...]) * c2 * g

    return pl.pallas_call(
        body,
        grid=(NB,),
        in_specs=[pl.BlockSpec((BE // 8, 8), lambda i: (i + OFFB, 0)),
                  pl.BlockSpec((BE // 2, 2 * R), lambda i: (i + OFFB, 0)),
                  pl.BlockSpec((2 * R, 2 * H), lambda i: (0, 0)),
                  pl.BlockSpec((1, 2 * H), lambda i: (0, 0)),
                  pl.BlockSpec((2 * ZP, 2 * H), lambda i: (0, 0))],
        out_specs=pl.BlockSpec((BE // 2, 2 * H), lambda i: (i, 0)),
        out_shape=jax.ShapeDtypeStruct((EH // 2, 2 * H), jnp.float32),
    )(v8, ea2, W2, b2p, emb2)


def _scatter_sc(msg, row, zeros_nh, NP, EH, H, ROFF):
    CH = 128
    NCHUNK = EH // CH
    NW = 32
    iters = (NCHUNK + NW - 1) // NW
    RPT = NP // 16
    mesh = plsc.VectorSubcoreMesh(core_axis_name="c", subcore_axis_name="s")

    @functools.partial(
        pl.kernel,
        out_type=jax.ShapeDtypeStruct((2, NP, H), jnp.float32),
        mesh=mesh,
        scratch_types=[pltpu.VMEM_SHARED((NP, H), jnp.float32),
                       pltpu.VMEM((2, CH, H), jnp.float32),
                       pltpu.VMEM((2, CH), jnp.int32),
                       pltpu.SemaphoreType.DMA,
                       pltpu.SemaphoreType.DMA],
        compiler_params=pltpu.CompilerParams(needs_layout_passes=False),
    )
    def k(msg_hbm, row_hbm, zero_hbm, out_hbm, aggr_sh, msg_v, idx_v, sem0, sem1):
        cid = lax.axis_index("c")
        sid = lax.axis_index("s")
        wid = cid * 16 + sid
        sems = (sem0, sem1)

        def start(t, b):
            off = (wid + t * NW) * CH
            pltpu.async_copy(row_hbm.at[pl.ds(ROFF + off, CH)], idx_v.at[b], sems[b])
            pltpu.async_copy(msg_hbm.at[pl.ds(off, CH)], msg_v.at[b], sems[b])

        def wait_and_scatter(b):
            pltpu.make_async_copy(row_hbm.at[pl.ds(0, CH)], idx_v.at[b], sems[b]).wait()
            pltpu.make_async_copy(msg_hbm.at[pl.ds(0, CH)], msg_v.at[b], sems[b]).wait()
            pltpu.sync_copy(msg_v.at[b], aggr_sh.at[idx_v.at[b]], add=True)

        pltpu.sync_copy(zero_hbm.at[pl.ds(sid * RPT, RPT)],
                        aggr_sh.at[pl.ds(sid * RPT, RPT)])
        plsc.subcore_barrier()
        start(0, 0)

        def body(tt, carry):
            for b in (0, 1):
                t = tt * 2 + b
                chunk = wid + t * NW

                @pl.when(wid + (t + 1) * NW < NCHUNK)
                def _():
                    start(t + 1, 1 - b)

                @pl.when(chunk < NCHUNK)
                def _():
                    wait_and_scatter(b)

            return carry

        lax.fori_loop(0, (iters + 1) // 2, body, 0)
        plsc.subcore_barrier()
        pltpu.sync_copy(aggr_sh.at[pl.ds(sid * RPT, RPT)],
                        out_hbm.at[cid, pl.ds(sid * RPT, RPT)])

    return k(msg, row, zeros_nh)


def _combine_tc(x, pa, pb, W1T, W2T, b2, N, H):
    BN = 400
    NB = N // BN

    def body(x_ref, pa0_ref, pa1_ref, pb0_ref, pb1_ref, w1_ref, w2_ref, b_ref,
             out_ref):
        aggr = (pa0_ref[0] + pa1_ref[0]) + (pb0_ref[0] + pb1_ref[0])
        out_ref[...] = (
            jnp.dot(x_ref[...], w1_ref[...], preferred_element_type=jnp.float32)
            + jnp.dot(aggr, w2_ref[...], preferred_element_type=jnp.float32)
            + b_ref[...])

    return pl.pallas_call(
        body,
        grid=(NB,),
        in_specs=[pl.BlockSpec((BN, H), lambda i: (i, 0)),
                  pl.BlockSpec((1, BN, H), lambda i: (0, i, 0)),
                  pl.BlockSpec((1, BN, H), lambda i: (1, i, 0)),
                  pl.BlockSpec((1, BN, H), lambda i: (0, i, 0)),
                  pl.BlockSpec((1, BN, H), lambda i: (1, i, 0)),
                  pl.BlockSpec((H, H), lambda i: (0, 0)),
                  pl.BlockSpec((H, H), lambda i: (0, 0)),
                  pl.BlockSpec((1, H), lambda i: (0, 0))],
        out_specs=pl.BlockSpec((BN, H), lambda i: (i, 0)),
        out_shape=jax.ShapeDtypeStruct((N, H), jnp.float32),
    )(x, pa, pa, pb, pb, W1T, W2T, b2)


def kernel(z, x, edge_index, edge_weight, edge_attr, emb, W_dist, b_dist, W_comb, b_comb):
    N, H = x.shape
    E = edge_weight.shape[0]
    R = edge_attr.shape[1]
    row = edge_index[0].astype(jnp.int32)
    col = edge_index[1].astype(jnp.int32)
    zcol = _zcol_sc(z.astype(jnp.int32), col, N, E)
    MZ = emb.shape[0]
    ZP = ((MZ + 127) // 128) * 128
    embp = jnp.zeros((ZP, H), emb.dtype).at[:MZ].set(emb)
    v = _encode_tc(edge_weight.reshape(E // 128, 128),
                   zcol.reshape(E // 128, 128), E // 128)
    NP = ((N + 127) // 128) * 128
    zeros_nh = jnp.zeros((NP, H), jnp.float32)
    v8 = v.reshape(E // 8, 8)
    WdT = W_dist.T
    b2 = b_dist.reshape(1, H)
    EH = E // 2
    OFFB = EH // 3200
    ea2 = edge_attr.reshape(E // 2, 2 * R)
    W2 = jnp.concatenate([jnp.pad(WdT, ((0, 0), (0, H))),
                          jnp.pad(WdT, ((0, 0), (H, 0)))], axis=0)
    b2p = jnp.concatenate([b_dist, b_dist]).reshape(1, 2 * H)
    emb2 = jnp.concatenate([jnp.pad(embp, ((0, 0), (0, H))),
                            jnp.pad(embp, ((0, 0), (H, 0)))], axis=0)
    msg_a = _msg_tc(v8, ea2, W2, b2p, emb2, EH, R, H, ZP, 0).reshape(EH, H)
    p_a = _scatter_sc(msg_a, row, zeros_nh, NP, EH, H, 0)
    msg_b = _msg_tc(v8, ea2, W2, b2p, emb2, EH, R, H, ZP, OFFB).reshape(EH, H)
    p_b = _scatter_sc(msg_b, row, zeros_nh, NP, EH, H, EH)
    return _combine_tc(x, p_a, p_b, W_comb[:, :H].T, W_comb[:, H:].T,
                       b_comb.reshape(1, H), N, H)


# speedup vs baseline: 1.4872x; 1.4872x over previous
import functools
import math

import jax
import jax.numpy as jnp
from jax import lax
from jax.experimental import pallas as pl
from jax.experimental.pallas import tpu as pltpu
from jax.experimental.pallas import tpu_sc as plsc

_CUTOFF = 5.0


def _zcol_sc(z, col, N, E):
    NW = 32
    EPW = E // NW
    mesh = plsc.VectorSubcoreMesh(core_axis_name="c", subcore_axis_name="s")

    @functools.partial(
        pl.kernel,
        out_type=jax.ShapeDtypeStruct((E,), jnp.int32),
        mesh=mesh,
        scratch_types=[pltpu.VMEM((N,), jnp.int32),
                       pltpu.VMEM((EPW,), jnp.int32),
                       pltpu.VMEM((EPW,), jnp.int32)],
        compiler_params=pltpu.CompilerParams(needs_layout_passes=False),
    )
    def k(z_hbm, col_hbm, zcol_hbm, z_v, col_v, out_v):
        wid = lax.axis_index("c") * 16 + lax.axis_index("s")
        base = wid * EPW
        pltpu.sync_copy(z_hbm, z_v)
        pltpu.sync_copy(col_hbm.at[pl.ds(base, EPW)], col_v)

        def body(j, carry):
            idx = col_v[pl.ds(j * 16, 16)]
            out_v[pl.ds(j * 16, 16)] = plsc.load_gather(z_v, [idx])
            return carry

        lax.fori_loop(0, EPW // 16, body, 0)
        pltpu.sync_copy(out_v, zcol_hbm.at[pl.ds(base, EPW)])

    return k(z, col)


def _encode_tc(ew128, zc128, EB):
    def body(ew_ref, zc_ref, v_ref):
        ew = ew_ref[...]
        c = 0.5 * (jnp.cos(ew * (math.pi / _CUTOFF)) + 1.0)
        c = jnp.where(ew < _CUTOFF, c, 0.0)
        v_ref[...] = zc_ref[...].astype(jnp.float32) + (0.25 + 0.5 * c)

    return pl.pallas_call(
        body,
        out_shape=jax.ShapeDtypeStruct((EB, 128), jnp.float32),
    )(ew128, zc128)


def _msg_tc(v8, ea, WdT, b2, embp, EH, R, H, ZP, OFFB):
    BE = 3200
    NB = EH // BE

    def body(v_ref, ea_ref, wd_ref, b_ref, emb_ref, out_ref):
        v = v_ref[...]
        v3 = lax.broadcast_in_dim(v, (BE // 8, 8, ZP), (0, 1))
        zcf3 = jnp.floor(v3)
        c2 = jnp.reshape((v3 - zcf3 - 0.25) * 2.0, (BE, ZP))
        onehot = jnp.reshape(
            (zcf3.astype(jnp.int32) ==
             lax.broadcasted_iota(jnp.int32, (BE // 8, 8, ZP), 2)
             ).astype(jnp.float32), (BE, ZP))
        mm = jnp.dot(ea_ref[...], wd_ref[...], preferred_element_type=jnp.float32)
        g = jnp.dot(onehot, emb_ref[...], preferred_element_type=jnp.float32)
        out_ref[...] = (mm + b_ref[...]) * c2 * g

    return pl.pallas_call(
        body,
        grid=(NB,),
        in_specs=[pl.BlockSpec((BE // 8, 8), lambda i: (i + OFFB, 0)),
                  pl.BlockSpec((BE, R), lambda i: (i + OFFB, 0)),
                  pl.BlockSpec((R, H), lambda i: (0, 0)),
                  pl.BlockSpec((1, H), lambda i: (0, 0)),
                  pl.BlockSpec((ZP, H), lambda i: (0, 0))],
        out_specs=pl.BlockSpec((BE, H), lambda i: (i, 0)),
        out_shape=jax.ShapeDtypeStruct((EH, H), jnp.float32),
    )(v8, ea, WdT, b2, embp)


def _scatter_sc(msg, row, zeros_nh, NP, EH, H, ROFF):
    CH = 128
    NCHUNK = EH // CH
    NW = 32
    iters = (NCHUNK + NW - 1) // NW
    RPT = NP // 16
    mesh = plsc.VectorSubcoreMesh(core_axis_name="c", subcore_axis_name="s")

    @functools.partial(
        pl.kernel,
        out_type=jax.ShapeDtypeStruct((2, NP, H), jnp.float32),
        mesh=mesh,
        scratch_types=[pltpu.VMEM_SHARED((NP, H), jnp.float32),
                       pltpu.VMEM((2, CH, H), jnp.float32),
                       pltpu.VMEM((2, CH), jnp.int32),
                       pltpu.SemaphoreType.DMA,
                       pltpu.SemaphoreType.DMA],
        compiler_params=pltpu.CompilerParams(needs_layout_passes=False),
    )
    def k(msg_hbm, row_hbm, zero_hbm, out_hbm, aggr_sh, msg_v, idx_v, sem0, sem1):
        cid = lax.axis_index("c")
        sid = lax.axis_index("s")
        wid = cid * 16 + sid
        sems = (sem0, sem1)

        def start(t, b):
            off = (wid + t * NW) * CH
            pltpu.async_copy(row_hbm.at[pl.ds(ROFF + off, CH)], idx_v.at[b], sems[b])
            pltpu.async_copy(msg_hbm.at[pl.ds(off, CH)], msg_v.at[b], sems[b])

        def wait_and_scatter(b):
            pltpu.make_async_copy(row_hbm.at[pl.ds(0, CH)], idx_v.at[b], sems[b]).wait()
            pltpu.make_async_copy(msg_hbm.at[pl.ds(0, CH)], msg_v.at[b], sems[b]).wait()
            pltpu.sync_copy(msg_v.at[b], aggr_sh.at[idx_v.at[b]], add=True)

        pltpu.sync_copy(zero_hbm.at[pl.ds(sid * RPT, RPT)],
                        aggr_sh.at[pl.ds(sid * RPT, RPT)])
        plsc.subcore_barrier()
        start(0, 0)

        def body(tt, carry):
            for b in (0, 1):
                t = tt * 2 + b
                chunk = wid + t * NW

                @pl.when(wid + (t + 1) * NW < NCHUNK)
                def _():
                    start(t + 1, 1 - b)

                @pl.when(chunk < NCHUNK)
                def _():
                    wait_and_scatter(b)

            return carry

        lax.fori_loop(0, (iters + 1) // 2, body, 0)
        plsc.subcore_barrier()
        pltpu.sync_copy(aggr_sh.at[pl.ds(sid * RPT, RPT)],
                        out_hbm.at[cid, pl.ds(sid * RPT, RPT)])

    return k(msg, row, zeros_nh)


def _combine_tc(x, pa, pb, W1T, W2T, b2, N, H):
    BN = 400
    NB = N // BN

    def body(x_ref, pa0_ref, pa1_ref, pb0_ref, pb1_ref, w1_ref, w2_ref, b_ref,
             out_ref):
        aggr = (pa0_ref[0] + pa1_ref[0]) + (pb0_ref[0] + pb1_ref[0])
        out_ref[...] = (
            jnp.dot(x_ref[...], w1_ref[...], preferred_element_type=jnp.float32)
            + jnp.dot(aggr, w2_ref[...], preferred_element_type=jnp.float32)
            + b_ref[...])

    return pl.pallas_call(
        body,
        grid=(NB,),
        in_specs=[pl.BlockSpec((BN, H), lambda i: (i, 0)),
                  pl.BlockSpec((1, BN, H), lambda i: (0, i, 0)),
                  pl.BlockSpec((1, BN, H), lambda i: (1, i, 0)),
                  pl.BlockSpec((1, BN, H), lambda i: (0, i, 0)),
                  pl.BlockSpec((1, BN, H), lambda i: (1, i, 0)),
                  pl.BlockSpec((H, H), lambda i: (0, 0)),
                  pl.BlockSpec((H, H), lambda i: (0, 0)),
                  pl.BlockSpec((1, H), lambda i: (0, 0))],
        out_specs=pl.BlockSpec((BN, H), lambda i: (i, 0)),
        out_shape=jax.ShapeDtypeStruct((N, H), jnp.float32),
    )(x, pa, pa, pb, pb, W1T, W2T, b2)


def kernel(z, x, edge_index, edge_weight, edge_attr, emb, W_dist, b_dist, W_comb, b_comb):
    N, H = x.shape
    E = edge_weight.shape[0]
    R = edge_attr.shape[1]
    row = edge_index[0].astype(jnp.int32)
    col = edge_index[1].astype(jnp.int32)
    zcol = _zcol_sc(z.astype(jnp.int32), col, N, E)
    MZ = emb.shape[0]
    ZP = ((MZ + 127) // 128) * 128
    embp = jnp.zeros((ZP, H), emb.dtype).at[:MZ].set(emb)
    v = _encode_tc(edge_weight.reshape(E // 128, 128),
                   zcol.reshape(E // 128, 128), E // 128)
    NP = ((N + 127) // 128) * 128
    zeros_nh = jnp.zeros((NP, H), jnp.float32)
    v8 = v.reshape(E // 8, 8)
    WdT = W_dist.T
    b2 = b_dist.reshape(1, H)
    EH = E // 2
    OFFB = EH // 3200
    msg_a = _msg_tc(v8, edge_attr, WdT, b2, embp, EH, R, H, ZP, 0)
    p_a = _scatter_sc(msg_a, row, zeros_nh, NP, EH, H, 0)
    msg_b = _msg_tc(v8, edge_attr, WdT, b2, embp, EH, R, H, ZP, OFFB)
    p_b = _scatter_sc(msg_b, row, zeros_nh, NP, EH, H, EH)
    return _combine_tc(x, p_a, p_b, W_comb[:, :H].T, W_comb[:, H:].T,
                       b_comb.reshape(1, H), N, H)
